# baseline (speedup 1.0000x reference)
import functools

import jax
import jax.numpy as jnp
import numpy as np
from jax import lax
from jax.experimental import pallas as pl
from jax.experimental.pallas import tpu as pltpu
from jax.experimental.pallas import tpu_sc as plsc

N = 10000
E = 640000
IN_DIM = 100
HID = 64
LAT = 64

NC = 2
NS = 16
NW = NC * NS

CHUNK = 128
CH = 157
EP = NW * CH * CHUNK
TOTCH = NW * CH
C0 = 140
C1 = 2 * CH - C0
NP = 10112
ROWS_PER_TILE = NP // NS

_MESH = plsc.VectorSubcoreMesh(core_axis_name="c", subcore_axis_name="s",
                               num_cores=NC, num_subcores=NS)
_SC_PARAMS = pltpu.CompilerParams(use_tc_tiling_on_sc=False)


def _zero_zbuf(zbuf, width):
    def body(i, _):
        for t in range(width // 16):
            zbuf[i, pl.ds(t * 16, 16)] = jnp.zeros((16,), jnp.float32)
        return _
    lax.fori_loop(0, CHUNK, body, None)


def _zero_acc_slice(zbuf, acc_sh, base):
    nfull = ROWS_PER_TILE // CHUNK
    rem = ROWS_PER_TILE - nfull * CHUNK
    for k in range(nfull):
        pltpu.sync_copy(zbuf, acc_sh.at[pl.ds(base + k * CHUNK, CHUNK)])
    if rem:
        pltpu.sync_copy(zbuf.at[pl.ds(0, rem)],
                        acc_sh.at[pl.ds(base + nfull * CHUNK, rem)])


@functools.partial(
    pl.kernel,
    out_type=jax.ShapeDtypeStruct((NC, 3, NP, 16), jnp.float32),
    mesh=_MESH,
    scratch_types=[
        pltpu.VMEM((CH, CHUNK), jnp.int32),
        pltpu.VMEM((CHUNK, 16), jnp.float32),
        pltpu.VMEM((CHUNK, 16), jnp.float32),
        pltpu.VMEM_SHARED((NP, 16), jnp.float32),
    ],
    compiler_params=_SC_PARAMS,
)
def _deg_kernel(dst_hbm, out_hbm, idx_v, ones_v, zbuf, acc_sh):
    c = lax.axis_index("c")
    s = lax.axis_index("s")
    wid = s * NC + c
    base = s * ROWS_PER_TILE

    def fill(i, _):
        ones_v[i, pl.ds(0, 16)] = jnp.ones((16,), jnp.float32)
        return _
    lax.fori_loop(0, CHUNK, fill, None)
    _zero_zbuf(zbuf, 16)

    for m in range(3):
        _zero_acc_slice(zbuf, acc_sh, base)
        plsc.subcore_barrier()
        pltpu.sync_copy(dst_hbm.at[m, pl.ds(wid * CH, CH)], idx_v)

        def body(j, _):
            pltpu.sync_copy(ones_v, acc_sh.at[idx_v.at[j]], add=True)
            return _
        lax.fori_loop(0, CH, body, None)

        plsc.subcore_barrier()
        pltpu.sync_copy(acc_sh.at[pl.ds(base, ROWS_PER_TILE)],
                        out_hbm.at[c, m, pl.ds(base, ROWS_PER_TILE)])


@functools.partial(
    pl.kernel,
    out_type=jax.ShapeDtypeStruct((NC, 3, NP, HID), jnp.float32),
    mesh=_MESH,
    scratch_types=[
        pltpu.VMEM((C1, CHUNK), jnp.int32),
        pltpu.VMEM((C1, CHUNK), jnp.int32),
        pltpu.VMEM((CHUNK, HID), jnp.float32),
        pltpu.VMEM((CHUNK, HID), jnp.float32),
        pltpu.VMEM((CHUNK, HID), jnp.float32),
        pltpu.VMEM_SHARED((NP, HID), jnp.float32),
        pltpu.SemaphoreType.DMA,
        pltpu.SemaphoreType.DMA,
        pltpu.SemaphoreType.DMA,
        pltpu.SemaphoreType.DMA,
    ],
    compiler_params=_SC_PARAMS,
)
def _agg_kernel(u_hbm, src_hbm, dst_hbm, out_hbm,
                idxs_v, idxd_v, rows0, rows1, zbuf, acc_sh,
                gsem0, gsem1, ssem0, ssem1):
    c = lax.axis_index("c")
    s = lax.axis_index("s")
    base = s * ROWS_PER_TILE
    nch = jnp.where(c == 0, C0, C1)

    _zero_zbuf(zbuf, HID)

    def gstart(j, buf, sem):
        pltpu.async_copy(u_hbm.at[idxs_v.at[j]], buf, sem)

    def gwait(buf, sem):
        pltpu.make_async_copy(u_hbm.at[idxs_v.at[0]], buf, sem).wait()

    def sstart(j, buf, sem):
        pltpu.async_copy(buf, acc_sh.at[idxd_v.at[j]], sem, add=True)

    def swait(buf, sem):
        pltpu.make_async_copy(buf, acc_sh.at[idxd_v.at[0]], sem).wait()

    for m in range(3):
        _zero_acc_slice(zbuf, acc_sh, base)
        plsc.subcore_barrier()

        @pl.when(c == 0)
        def _():
            pltpu.sync_copy(src_hbm.at[m, pl.ds(s * C0, C0)],
                            idxs_v.at[pl.ds(0, C0)])
            pltpu.sync_copy(dst_hbm.at[m, pl.ds(s * C0, C0)],
                            idxd_v.at[pl.ds(0, C0)])

        @pl.when(c == 1)
        def _():
            pltpu.sync_copy(src_hbm.at[m, pl.ds(NS * C0 + s * C1, C1)],
                            idxs_v)
            pltpu.sync_copy(dst_hbm.at[m, pl.ds(NS * C0 + s * C1, C1)],
                            idxd_v)

        gstart(0, rows0, gsem0)

        def body(k, _):
            j0 = 2 * k
            gwait(rows0, gsem0)
            gstart(j0 + 1, rows1, gsem1)
            sstart(j0, rows0, ssem0)
            gwait(rows1, gsem1)
            swait(rows0, ssem0)
            gstart(j0 + 2, rows0, gsem0)
            sstart(j0 + 1, rows1, ssem1)
            swait(rows1, ssem1)
            return _
        lax.fori_loop(0, nch // 2 - 1, body, None)
        jA = nch - 2
        gwait(rows0, gsem0)
        gstart(jA + 1, rows1, gsem1)
        sstart(jA, rows0, ssem0)
        gwait(rows1, gsem1)
        swait(rows0, ssem0)
        sstart(jA + 1, rows1, ssem1)
        swait(rows1, ssem1)

        plsc.subcore_barrier()
        pltpu.sync_copy(acc_sh.at[pl.ds(base, ROWS_PER_TILE)],
                        out_hbm.at[c, m, pl.ds(base, ROWS_PER_TILE)])


BN = 1000
NB = N // BN


def _dinv_from(dp_ref):
    deg = dp_ref[0, :, 0:1] + dp_ref[1, :, 0:1] + 1.0
    return lax.rsqrt(deg)


def _stage_a_body(x_ref, w1_ref, dp_ref, u1_ref):
    dinv = _dinv_from(dp_ref)
    xw = jnp.dot(x_ref[...], w1_ref[...], preferred_element_type=jnp.float32)
    u1_ref[...] = dinv * xw


def _stage_a(x3, w1_3, dp):
    return pl.pallas_call(
        _stage_a_body,
        grid=(3, NB),
        in_specs=[
            pl.BlockSpec((None, BN, IN_DIM), lambda m, i: (m, i, 0)),
            pl.BlockSpec((None, IN_DIM, HID), lambda m, i: (m, 0, 0)),
            pl.BlockSpec((2, None, BN, 16), lambda m, i: (0, m, i, 0)),
        ],
        out_specs=pl.BlockSpec((None, BN, HID), lambda m, i: (m, i, 0)),
        out_shape=jax.ShapeDtypeStruct((3, N, HID), jnp.float32),
    )(x3, w1_3, dp)


def _stage_b_body(p_ref, u1_ref, dp_ref, w2_ref, b1_ref, u2_ref):
    dinv = _dinv_from(dp_ref)
    h1 = dinv * (p_ref[0] + p_ref[1] + u1_ref[...]) + b1_ref[...]
    h1 = jnp.maximum(h1, 0.0)
    u2_ref[...] = dinv * jnp.dot(h1, w2_ref[...],
                                 preferred_element_type=jnp.float32)


def _stage_b(p1, u1, dp, w2_3, b1_3):
    return pl.pallas_call(
        _stage_b_body,
        grid=(3, NB),
        in_specs=[
            pl.BlockSpec((2, None, BN, HID), lambda m, i: (0, m, i, 0)),
            pl.BlockSpec((None, BN, HID), lambda m, i: (m, i, 0)),
            pl.BlockSpec((2, None, BN, 16), lambda m, i: (0, m, i, 0)),
            pl.BlockSpec((None, HID, LAT), lambda m, i: (m, 0, 0)),
            pl.BlockSpec((None, 1, HID), lambda m, i: (m, 0, 0)),
        ],
        out_specs=pl.BlockSpec((None, BN, LAT), lambda m, i: (m, i, 0)),
        out_shape=jax.ShapeDtypeStruct((3, N, LAT), jnp.float32),
    )(p1, u1, dp, w2_3, b1_3)


def _stage_c_body(p_ref, u2_ref, dp_ref, b2_ref, wmu_ref, bmu_ref,
                  wlv_ref, blv_ref, eps_ref, mu_ref, lv_ref, z_ref):
    dinv = _dinv_from(dp_ref)
    h = dinv * (p_ref[0] + p_ref[1] + u2_ref[...]) + b2_ref[...]
    mu = jnp.dot(h, wmu_ref[...], preferred_element_type=jnp.float32) \
        + bmu_ref[...]
    lv = jnp.dot(h, wlv_ref[...], preferred_element_type=jnp.float32) \
        + blv_ref[...]
    z = mu + eps_ref[...] * jnp.exp(0.5 * lv)
    mu_ref[...] = mu
    lv_ref[...] = lv
    z_ref[...] = z


def _stage_c(p2, u2, dp, b2_3, wmu, bmu, wlv, blv, eps):
    sds = jax.ShapeDtypeStruct((3, N, LAT), jnp.float32)
    return pl.pallas_call(
        _stage_c_body,
        grid=(3, NB),
        in_specs=[
            pl.BlockSpec((2, None, BN, HID), lambda m, i: (0, m, i, 0)),
            pl.BlockSpec((None, BN, LAT), lambda m, i: (m, i, 0)),
            pl.BlockSpec((2, None, BN, 16), lambda m, i: (0, m, i, 0)),
            pl.BlockSpec((None, 1, LAT), lambda m, i: (m, 0, 0)),
            pl.BlockSpec((LAT, LAT), lambda m, i: (0, 0)),
            pl.BlockSpec((1, LAT), lambda m, i: (0, 0)),
            pl.BlockSpec((LAT, LAT), lambda m, i: (0, 0)),
            pl.BlockSpec((1, LAT), lambda m, i: (0, 0)),
            pl.BlockSpec((None, BN, LAT), lambda m, i: (m, i, 0)),
        ],
        out_specs=[
            pl.BlockSpec((None, BN, LAT), lambda m, i: (m, i, 0)),
            pl.BlockSpec((None, BN, LAT), lambda m, i: (m, i, 0)),
            pl.BlockSpec((None, BN, LAT), lambda m, i: (m, i, 0)),
        ],
        out_shape=[sds, sds, sds],
    )(p2, u2, dp, b2_3, wmu, bmu, wlv, blv, eps)


BND = 2000
BC = 2048
NBD = N // BND
NBC = -(-N // BC)


def _stage_d_body(z_ref, wd_ref, bd_ref, r_ref):
    acc = jnp.dot(z_ref[...], wd_ref[...],
                  preferred_element_type=jnp.float32) + bd_ref[...]
    r_ref[...] = jnp.tanh(acc)


def _stage_d(z, wd, bd):
    return pl.pallas_call(
        _stage_d_body,
        grid=(NBD, NBC),
        in_specs=[
            pl.BlockSpec((BND, LAT), lambda i, j: (i, 0)),
            pl.BlockSpec((LAT, BC), lambda i, j: (0, j)),
            pl.BlockSpec((1, BC), lambda i, j: (0, j)),
        ],
        out_specs=pl.BlockSpec((BND, BC), lambda i, j: (i, j)),
        out_shape=jax.ShapeDtypeStruct((N, N), jnp.float32),
    )(z, wd, bd)


def _prep_edges(ei_s, ei_i, ei_d):
    pad = EP - E
    srcs, dsts = [], []
    for m, ei in enumerate((ei_s, ei_i, ei_d)):
        src = jnp.concatenate(
            [ei[0] + m * N, jnp.full((pad,), m * N, jnp.int32)])
        dst = jnp.concatenate([ei[1], jnp.full((pad,), N, jnp.int32)])
        srcs.append(src)
        dsts.append(dst)
    src3 = jnp.stack(srcs).reshape(3, TOTCH, CHUNK)
    dst3 = jnp.stack(dsts).reshape(3, TOTCH, CHUNK)
    return src3, dst3


def kernel(x_s, x_i, x_d, ei_s, ei_i, ei_d, params):
    p = params
    x3 = jnp.stack([x_s, x_i, x_d])
    w1_3 = jnp.stack([p['enc_s']['W1'], p['enc_i']['W1'], p['enc_d']['W1']])
    b1_3 = jnp.stack([p['enc_s']['b1'], p['enc_i']['b1'],
                      p['enc_d']['b1']])[:, None, :]
    w2_3 = jnp.stack([p['enc_s']['W2'], p['enc_i']['W2'], p['enc_d']['W2']])
    b2_3 = jnp.stack([p['enc_s']['b2'], p['enc_i']['b2'],
                      p['enc_d']['b2']])[:, None, :]
    eps = jax.random.normal(jax.random.key(42), (N, 3, LAT), jnp.float32)
    eps = jnp.transpose(eps, (1, 0, 2))

    src3, dst3 = _prep_edges(ei_s, ei_i, ei_d)

    dp = _deg_kernel(dst3)
    u1 = _stage_a(x3, w1_3, dp)
    p1 = _agg_kernel(u1.reshape(3 * N, HID), src3, dst3)
    u2 = _stage_b(p1, u1, dp, w2_3, b1_3)
    p2 = _agg_kernel(u2.reshape(3 * N, HID), src3, dst3)
    mu3, lv3, z3 = _stage_c(p2, u2, dp, b2_3, p['Wmu'],
                            p['bmu'][None, :], p['Wlv'],
                            p['blv'][None, :], eps)
    rs = _stage_d(z3[0], p['Wds'], p['bds'][None, :])
    ri = _stage_d(z3[1], p['Wdi'], p['bdi'][None, :])
    rd = _stage_d(z3[2], p['Wdd'], p['bdd'][None, :])
    mu = jnp.transpose(mu3, (1, 0, 2))
    logvar = jnp.transpose(lv3, (1, 0, 2))
    return (rs, ri, rd, mu, logvar)


# speedup vs baseline: 1.1131x; 1.1131x over previous
import functools

import jax
import jax.numpy as jnp
import numpy as np
from jax import lax
from jax.experimental import pallas as pl
from jax.experimental.pallas import tpu as pltpu
from jax.experimental.pallas import tpu_sc as plsc

N = 10000
E = 640000
IN_DIM = 100
HID = 64
LAT = 64

NC = 2
NS = 16
NW = NC * NS

CHUNK = 128
CH = 157
EP = NW * CH * CHUNK
TOTCH = NW * CH
C0 = 174
C1 = 2 * CH - C0
CMAX = max(C0, C1)
NP = 10112
ROWS_PER_TILE = NP // NS

_MESH = plsc.VectorSubcoreMesh(core_axis_name="c", subcore_axis_name="s",
                               num_cores=NC, num_subcores=NS)
_SC_PARAMS = pltpu.CompilerParams(use_tc_tiling_on_sc=False)


def _zero_zbuf(zbuf, width):
    def body(i, _):
        for t in range(width // 16):
            zbuf[i, pl.ds(t * 16, 16)] = jnp.zeros((16,), jnp.float32)
        return _
    lax.fori_loop(0, CHUNK, body, None)


def _zero_acc_slice(zbuf, acc_sh, base):
    nfull = ROWS_PER_TILE // CHUNK
    rem = ROWS_PER_TILE - nfull * CHUNK
    for k in range(nfull):
        pltpu.sync_copy(zbuf, acc_sh.at[pl.ds(base + k * CHUNK, CHUNK)])
    if rem:
        pltpu.sync_copy(zbuf.at[pl.ds(0, rem)],
                        acc_sh.at[pl.ds(base + nfull * CHUNK, rem)])


@functools.partial(
    pl.kernel,
    out_type=jax.ShapeDtypeStruct((NC, 3, NP, 16), jnp.float32),
    mesh=_MESH,
    scratch_types=[
        pltpu.VMEM((CH, CHUNK), jnp.int32),
        pltpu.VMEM((CHUNK, 16), jnp.float32),
        pltpu.VMEM((CHUNK, 16), jnp.float32),
        pltpu.VMEM_SHARED((NP, 16), jnp.float32),
    ],
    compiler_params=_SC_PARAMS,
)
def _deg_kernel(dst_hbm, out_hbm, idx_v, ones_v, zbuf, acc_sh):
    c = lax.axis_index("c")
    s = lax.axis_index("s")
    wid = s * NC + c
    base = s * ROWS_PER_TILE

    def fill(i, _):
        ones_v[i, pl.ds(0, 16)] = jnp.ones((16,), jnp.float32)
        return _
    lax.fori_loop(0, CHUNK, fill, None)
    _zero_zbuf(zbuf, 16)

    for m in range(3):
        _zero_acc_slice(zbuf, acc_sh, base)
        plsc.subcore_barrier()
        pltpu.sync_copy(dst_hbm.at[m, pl.ds(wid * CH, CH)], idx_v)

        def body(j, _):
            pltpu.sync_copy(ones_v, acc_sh.at[idx_v.at[j]], add=True)
            return _
        lax.fori_loop(0, CH, body, None)

        plsc.subcore_barrier()
        pltpu.sync_copy(acc_sh.at[pl.ds(base, ROWS_PER_TILE)],
                        out_hbm.at[c, m, pl.ds(base, ROWS_PER_TILE)])


@functools.partial(
    pl.kernel,
    out_type=jax.ShapeDtypeStruct((NC, 3, NP, HID), jnp.float32),
    mesh=_MESH,
    scratch_types=[
        pltpu.VMEM((CMAX, CHUNK), jnp.int32),
        pltpu.VMEM((CMAX, CHUNK), jnp.int32),
        pltpu.VMEM((CHUNK, HID), jnp.float32),
        pltpu.VMEM((CHUNK, HID), jnp.float32),
        pltpu.VMEM((CHUNK, HID), jnp.float32),
        pltpu.VMEM_SHARED((NP, HID), jnp.float32),
        pltpu.SemaphoreType.DMA,
        pltpu.SemaphoreType.DMA,
        pltpu.SemaphoreType.DMA,
        pltpu.SemaphoreType.DMA,
    ],
    compiler_params=_SC_PARAMS,
)
def _agg_kernel(u_hbm, src_hbm, dst_hbm, out_hbm,
                idxs_v, idxd_v, rows0, rows1, zbuf, acc_sh,
                gsem0, gsem1, ssem0, ssem1):
    c = lax.axis_index("c")
    s = lax.axis_index("s")
    base = s * ROWS_PER_TILE
    nch = jnp.where(c == 0, C0, C1)

    _zero_zbuf(zbuf, HID)

    def gstart(j, buf, sem):
        pltpu.async_copy(u_hbm.at[idxs_v.at[j]], buf, sem)

    def gwait(buf, sem):
        pltpu.make_async_copy(u_hbm.at[idxs_v.at[0]], buf, sem).wait()

    def sstart(j, buf, sem):
        pltpu.async_copy(buf, acc_sh.at[idxd_v.at[j]], sem, add=True)

    def swait(buf, sem):
        pltpu.make_async_copy(buf, acc_sh.at[idxd_v.at[0]], sem).wait()

    for m in range(3):
        _zero_acc_slice(zbuf, acc_sh, base)
        plsc.subcore_barrier()

        @pl.when(c == 0)
        def _():
            pltpu.sync_copy(src_hbm.at[m, pl.ds(s * C0, C0)],
                            idxs_v.at[pl.ds(0, C0)])
            pltpu.sync_copy(dst_hbm.at[m, pl.ds(s * C0, C0)],
                            idxd_v.at[pl.ds(0, C0)])

        @pl.when(c == 1)
        def _():
            pltpu.sync_copy(src_hbm.at[m, pl.ds(NS * C0 + s * C1, C1)],
                            idxs_v.at[pl.ds(0, C1)])
            pltpu.sync_copy(dst_hbm.at[m, pl.ds(NS * C0 + s * C1, C1)],
                            idxd_v.at[pl.ds(0, C1)])

        gstart(0, rows0, gsem0)

        def body(k, _):
            j0 = 2 * k
            gwait(rows0, gsem0)
            gstart(j0 + 1, rows1, gsem1)
            sstart(j0, rows0, ssem0)
            gwait(rows1, gsem1)
            swait(rows0, ssem0)
            gstart(j0 + 2, rows0, gsem0)
            sstart(j0 + 1, rows1, ssem1)
            swait(rows1, ssem1)
            return _
        lax.fori_loop(0, nch // 2 - 1, body, None)
        jA = nch - 2
        gwait(rows0, gsem0)
        gstart(jA + 1, rows1, gsem1)
        sstart(jA, rows0, ssem0)
        gwait(rows1, gsem1)
        swait(rows0, ssem0)
        sstart(jA + 1, rows1, ssem1)
        swait(rows1, ssem1)

        plsc.subcore_barrier()
        pltpu.sync_copy(acc_sh.at[pl.ds(base, ROWS_PER_TILE)],
                        out_hbm.at[c, m, pl.ds(base, ROWS_PER_TILE)])


BN = 1000
NB = N // BN


def _dinv_from(dp_ref):
    deg = dp_ref[0, :, 0:1] + dp_ref[1, :, 0:1] + 1.0
    return lax.rsqrt(deg)


def _stage_a_body(x_ref, w1_ref, dp_ref, u1_ref):
    dinv = _dinv_from(dp_ref)
    xw = jnp.dot(x_ref[...], w1_ref[...], preferred_element_type=jnp.float32)
    u1_ref[...] = dinv * xw


def _stage_a(x3, w1_3, dp):
    return pl.pallas_call(
        _stage_a_body,
        grid=(3, NB),
        in_specs=[
            pl.BlockSpec((None, BN, IN_DIM), lambda m, i: (m, i, 0)),
            pl.BlockSpec((None, IN_DIM, HID), lambda m, i: (m, 0, 0)),
            pl.BlockSpec((2, None, BN, 16), lambda m, i: (0, m, i, 0)),
        ],
        out_specs=pl.BlockSpec((None, BN, HID), lambda m, i: (m, i, 0)),
        out_shape=jax.ShapeDtypeStruct((3, N, HID), jnp.float32),
    )(x3, w1_3, dp)


def _stage_b_body(p_ref, u1_ref, dp_ref, w2_ref, b1_ref, u2_ref):
    dinv = _dinv_from(dp_ref)
    h1 = dinv * (p_ref[0] + p_ref[1] + u1_ref[...]) + b1_ref[...]
    h1 = jnp.maximum(h1, 0.0)
    u2_ref[...] = dinv * jnp.dot(h1, w2_ref[...],
                                 preferred_element_type=jnp.float32)


def _stage_b(p1, u1, dp, w2_3, b1_3):
    return pl.pallas_call(
        _stage_b_body,
        grid=(3, NB),
        in_specs=[
            pl.BlockSpec((2, None, BN, HID), lambda m, i: (0, m, i, 0)),
            pl.BlockSpec((None, BN, HID), lambda m, i: (m, i, 0)),
            pl.BlockSpec((2, None, BN, 16), lambda m, i: (0, m, i, 0)),
            pl.BlockSpec((None, HID, LAT), lambda m, i: (m, 0, 0)),
            pl.BlockSpec((None, 1, HID), lambda m, i: (m, 0, 0)),
        ],
        out_specs=pl.BlockSpec((None, BN, LAT), lambda m, i: (m, i, 0)),
        out_shape=jax.ShapeDtypeStruct((3, N, LAT), jnp.float32),
    )(p1, u1, dp, w2_3, b1_3)


def _stage_c_body(p_ref, u2_ref, dp_ref, b2_ref, wmu_ref, bmu_ref,
                  wlv_ref, blv_ref, eps_ref, mu_ref, lv_ref, z_ref):
    dinv = _dinv_from(dp_ref)
    h = dinv * (p_ref[0] + p_ref[1] + u2_ref[...]) + b2_ref[...]
    mu = jnp.dot(h, wmu_ref[...], preferred_element_type=jnp.float32) \
        + bmu_ref[...]
    lv = jnp.dot(h, wlv_ref[...], preferred_element_type=jnp.float32) \
        + blv_ref[...]
    z = mu + eps_ref[...] * jnp.exp(0.5 * lv)
    mu_ref[...] = mu
    lv_ref[...] = lv
    z_ref[...] = z


def _stage_c(p2, u2, dp, b2_3, wmu, bmu, wlv, blv, eps):
    sds = jax.ShapeDtypeStruct((3, N, LAT), jnp.float32)
    return pl.pallas_call(
        _stage_c_body,
        grid=(3, NB),
        in_specs=[
            pl.BlockSpec((2, None, BN, HID), lambda m, i: (0, m, i, 0)),
            pl.BlockSpec((None, BN, LAT), lambda m, i: (m, i, 0)),
            pl.BlockSpec((2, None, BN, 16), lambda m, i: (0, m, i, 0)),
            pl.BlockSpec((None, 1, LAT), lambda m, i: (m, 0, 0)),
            pl.BlockSpec((LAT, LAT), lambda m, i: (0, 0)),
            pl.BlockSpec((1, LAT), lambda m, i: (0, 0)),
            pl.BlockSpec((LAT, LAT), lambda m, i: (0, 0)),
            pl.BlockSpec((1, LAT), lambda m, i: (0, 0)),
            pl.BlockSpec((None, BN, LAT), lambda m, i: (m, i, 0)),
        ],
        out_specs=[
            pl.BlockSpec((None, BN, LAT), lambda m, i: (m, i, 0)),
            pl.BlockSpec((None, BN, LAT), lambda m, i: (m, i, 0)),
            pl.BlockSpec((None, BN, LAT), lambda m, i: (m, i, 0)),
        ],
        out_shape=[sds, sds, sds],
    )(p2, u2, dp, b2_3, wmu, bmu, wlv, blv, eps)


BND = 2000
BC = 2048
NBD = N // BND
NBC = -(-N // BC)


def _stage_d_body(z_ref, wd_ref, bd_ref, r_ref):
    acc = jnp.dot(z_ref[...], wd_ref[...],
                  preferred_element_type=jnp.float32) + bd_ref[...]
    r_ref[...] = jnp.tanh(acc)


def _stage_d(z, wd, bd):
    return pl.pallas_call(
        _stage_d_body,
        grid=(NBD, NBC),
        in_specs=[
            pl.BlockSpec((BND, LAT), lambda i, j: (i, 0)),
            pl.BlockSpec((LAT, BC), lambda i, j: (0, j)),
            pl.BlockSpec((1, BC), lambda i, j: (0, j)),
        ],
        out_specs=pl.BlockSpec((BND, BC), lambda i, j: (i, j)),
        out_shape=jax.ShapeDtypeStruct((N, N), jnp.float32),
    )(z, wd, bd)


def _prep_edges(ei_s, ei_i, ei_d):
    pad = EP - E
    srcs, dsts = [], []
    for m, ei in enumerate((ei_s, ei_i, ei_d)):
        src = jnp.concatenate(
            [ei[0] + m * N, jnp.full((pad,), m * N, jnp.int32)])
        dst = jnp.concatenate([ei[1], jnp.full((pad,), N, jnp.int32)])
        srcs.append(src)
        dsts.append(dst)
    src3 = jnp.stack(srcs).reshape(3, TOTCH, CHUNK)
    dst3 = jnp.stack(dsts).reshape(3, TOTCH, CHUNK)
    return src3, dst3


def kernel(x_s, x_i, x_d, ei_s, ei_i, ei_d, params):
    p = params
    x3 = jnp.stack([x_s, x_i, x_d])
    w1_3 = jnp.stack([p['enc_s']['W1'], p['enc_i']['W1'], p['enc_d']['W1']])
    b1_3 = jnp.stack([p['enc_s']['b1'], p['enc_i']['b1'],
                      p['enc_d']['b1']])[:, None, :]
    w2_3 = jnp.stack([p['enc_s']['W2'], p['enc_i']['W2'], p['enc_d']['W2']])
    b2_3 = jnp.stack([p['enc_s']['b2'], p['enc_i']['b2'],
                      p['enc_d']['b2']])[:, None, :]
    eps = jax.random.normal(jax.random.key(42), (N, 3, LAT), jnp.float32)
    eps = jnp.transpose(eps, (1, 0, 2))

    src3, dst3 = _prep_edges(ei_s, ei_i, ei_d)

    dp = _deg_kernel(dst3)
    u1 = _stage_a(x3, w1_3, dp)
    p1 = _agg_kernel(u1.reshape(3 * N, HID), src3, dst3)
    u2 = _stage_b(p1, u1, dp, w2_3, b1_3)
    p2 = _agg_kernel(u2.reshape(3 * N, HID), src3, dst3)
    mu3, lv3, z3 = _stage_c(p2, u2, dp, b2_3, p['Wmu'],
                            p['bmu'][None, :], p['Wlv'],
                            p['blv'][None, :], eps)
    rs = _stage_d(z3[0], p['Wds'], p['bds'][None, :])
    ri = _stage_d(z3[1], p['Wdi'], p['bdi'][None, :])
    rd = _stage_d(z3[2], p['Wdd'], p['bdd'][None, :])
    mu = jnp.transpose(mu3, (1, 0, 2))
    logvar = jnp.transpose(lv3, (1, 0, 2))
    return (rs, ri, rd, mu, logvar)
